# baseline (speedup 1.0000x reference)
import functools

import jax
import jax.numpy as jnp
from jax import lax
from jax.experimental import pallas as pl
from jax.experimental.pallas import tpu as pltpu
from jax.experimental.pallas import tpu_sc as plsc

_E = 320000
_NW = 32
_MAIN = 9984
_REM = _E - _MAIN * _NW
_FLAT = _MAIN + _REM
_NPAD = 10240
_ZCH = _NPAD // 16
_DUMP = _NPAD - 2


def _sc_histograms(ei):
  mesh = plsc.VectorSubcoreMesh(core_axis_name="c", subcore_axis_name="s")

  @functools.partial(
      pl.kernel,
      out_type=(
          jax.ShapeDtypeStruct((2, _NPAD), jnp.int32),
          jax.ShapeDtypeStruct((2, _NPAD), jnp.int32),
      ),
      mesh=mesh,
      scratch_types=[
          pltpu.VMEM((_FLAT,), jnp.int32),
          pltpu.VMEM((_FLAT,), jnp.int32),
          pltpu.VMEM((_FLAT,), jnp.int32),
          pltpu.VMEM((_FLAT,), jnp.int32),
          pltpu.VMEM((_ZCH,), jnp.int32),
          pltpu.VMEM_SHARED((_NPAD,), jnp.int32),
          pltpu.VMEM_SHARED((_NPAD,), jnp.int32),
          pltpu.SemaphoreType.DMA,
          pltpu.SemaphoreType.DMA,
      ],
  )
  def hist_kernel(ei_hbm, out_indeg, out_cnt0,
                  src_v, dst_v, ones_v, val_v, zero_v,
                  hist_d, hist_c, sem_a, sem_b):
    c = lax.axis_index("c")
    s = lax.axis_index("s")
    wid = s * 2 + c
    base = wid * _MAIN
    half = _MAIN // 2

    cp_d1 = pltpu.async_copy(
        ei_hbm.at[1, pl.ds(base, half)], dst_v.at[pl.ds(0, half)], sem_b)
    cp_d2 = pltpu.async_copy(
        ei_hbm.at[1, pl.ds(base + half, half)],
        dst_v.at[pl.ds(half, half)], sem_b)
    cp_s = pltpu.async_copy(
        ei_hbm.at[0, pl.ds(base, _MAIN)], src_v.at[pl.ds(0, _MAIN)], sem_a)

    @pl.when(wid == _NW - 1)
    def _():
      pltpu.sync_copy(ei_hbm.at[0, pl.ds(_MAIN * _NW, _REM)],
                      src_v.at[pl.ds(_MAIN, _REM)])
      pltpu.sync_copy(ei_hbm.at[1, pl.ds(_MAIN * _NW, _REM)],
                      dst_v.at[pl.ds(_MAIN, _REM)])

    zero16 = jnp.full((16,), 0, jnp.int32)
    one16 = jnp.full((16,), 1, jnp.int32)

    @pl.when(wid != _NW - 1)
    def _():
      dump16 = jnp.full((16,), _DUMP, jnp.int32)

      def pad_body(i, carry):
        sl = pl.ds(_MAIN + i * 16, 16)
        src_v[sl] = zero16
        dst_v[sl] = dump16
        return carry

      lax.fori_loop(0, _REM // 16, pad_body, 0)

    def zero_body(i, carry):
      zero_v[pl.ds(i * 16, 16)] = zero16
      return carry

    lax.fori_loop(0, _ZCH // 16, zero_body, 0)
    pltpu.sync_copy(zero_v, hist_d.at[pl.ds(s * _ZCH, _ZCH)])
    pltpu.sync_copy(zero_v, hist_c.at[pl.ds(s * _ZCH, _ZCH)])

    def ones_body(i, carry):
      ones_v[pl.ds(i * 16, 16)] = one16
      return carry

    lax.fori_loop(0, _FLAT // 16, ones_body, 0)

    cp_d1.wait()
    plsc.subcore_barrier()

    sc_d1 = pltpu.async_copy(
        ones_v.at[pl.ds(0, half)],
        hist_d.at[dst_v.at[pl.ds(0, half)]], sem_b, add=True)

    cp_d2.wait()

    def val_body(i, carry):
      sl = pl.ds(i * 16, 16)
      val_v[sl] = jnp.where(dst_v[sl] == 0, 1, 0).astype(jnp.int32)
      return carry

    lax.fori_loop(0, _FLAT // 16, val_body, 0)

    sc_d2 = pltpu.async_copy(
        ones_v.at[pl.ds(half, _FLAT - half)],
        hist_d.at[dst_v.at[pl.ds(half, _FLAT - half)]], sem_b, add=True)
    cp_s.wait()
    sc_c = pltpu.async_copy(val_v, hist_c.at[src_v], sem_a, add=True)
    sc_d1.wait()
    sc_d2.wait()
    sc_c.wait()
    plsc.subcore_barrier()

    @pl.when(s == 0)
    def _():
      pltpu.sync_copy(hist_d, out_indeg.at[c])
      pltpu.sync_copy(hist_c, out_cnt0.at[c])

  return hist_kernel(ei)


def _tc_dense(ind2, cnt2, x, w_emb, b_emb2, w_gcn, b_gcn2, w_cls, b_cls2):
  nb = x.shape[0]
  d = x.shape[1]

  def body(ind_ref, cnt_ref, x_ref, we_ref, be_ref, wg_ref, bg_ref,
           wc_ref, bc_ref, o_ref):
    xx = x_ref[:]
    e0 = jnp.maximum(
        jnp.dot(xx, we_ref[:], preferred_element_type=jnp.float32)
        + be_ref[:], 0.0)
    h1 = jnp.dot(e0, wg_ref[:], preferred_element_type=jnp.float32)
    bg = bg_ref[:]
    h2 = jnp.concatenate([h1[b:b + 1, :] for b in range(nb)], axis=1)
    bg2 = jnp.concatenate([bg] * nb, axis=1)
    h2_hi = h2.astype(jnp.bfloat16).astype(jnp.float32)
    h2_lo = h2 - h2_hi

    ind_full = (ind_ref[0:1, :] + ind_ref[1:2, :]).astype(jnp.float32)
    cnt_full = (cnt_ref[0:1, :] + cnt_ref[1:2, :]).astype(jnp.float32)
    dn = (((0,), (0,)), ((), ()))
    outer = (
        lax.dot_general(ind_full, h2_hi, dn,
                        preferred_element_type=jnp.float32)
        + lax.dot_general(ind_full, h2_lo, dn,
                          preferred_element_type=jnp.float32))
    z = jnp.maximum(outer + bg2, 0.0)
    acc = jnp.dot(cnt_full, z, preferred_element_type=jnp.float32)

    sm = jnp.concatenate(
        [acc[:, b * 128:(b + 1) * 128] for b in range(nb)], axis=0)
    out2 = jnp.maximum(
        jnp.dot(sm, wg_ref[:], preferred_element_type=jnp.float32) + bg, 0.0)
    y = jnp.dot(out2, wc_ref[:], preferred_element_type=jnp.float32) + bc_ref[:]
    o_ref[:] = y

  return pl.pallas_call(
      body,
      out_shape=jax.ShapeDtypeStruct((nb, 1), jnp.float32),
  )(ind2, cnt2, x, w_emb, b_emb2, w_gcn, b_gcn2, w_cls, b_cls2)


def kernel(x, edge_index, W_emb, b_emb, W_gcn, b_gcn, W_cls, b_cls):
  ei = edge_index.astype(jnp.int32)
  ind2, cnt2 = _sc_histograms(ei)
  d = x.shape[1]
  return _tc_dense(
      ind2, cnt2, x, W_emb, b_emb.reshape(1, d), W_gcn,
      b_gcn.reshape(1, d), W_cls, b_cls.reshape(1, 1))


# speedup vs baseline: 1.0187x; 1.0187x over previous
import functools

import jax
import jax.numpy as jnp
from jax import lax
from jax.experimental import pallas as pl
from jax.experimental.pallas import tpu as pltpu
from jax.experimental.pallas import tpu_sc as plsc

_E = 320000
_NW = 32
_MAIN = 9984
_REM = _E - _MAIN * _NW
_FLAT = _MAIN + _REM
_NPAD = 10240
_ZCH = _NPAD // 16
_DUMP = _NPAD - 2


def _sc_histograms(ei):
  mesh = plsc.VectorSubcoreMesh(core_axis_name="c", subcore_axis_name="s")

  @functools.partial(
      pl.kernel,
      out_type=(
          jax.ShapeDtypeStruct((2, _NPAD), jnp.int32),
          jax.ShapeDtypeStruct((2, _NPAD), jnp.int32),
      ),
      mesh=mesh,
      scratch_types=[
          pltpu.VMEM((_FLAT,), jnp.int32),
          pltpu.VMEM((_FLAT,), jnp.int32),
          pltpu.VMEM((_FLAT,), jnp.int32),
          pltpu.VMEM((_FLAT,), jnp.int32),
          pltpu.VMEM((_ZCH,), jnp.int32),
          pltpu.VMEM_SHARED((_NPAD,), jnp.int32),
          pltpu.VMEM_SHARED((_NPAD,), jnp.int32),
          pltpu.SemaphoreType.DMA,
          pltpu.SemaphoreType.DMA,
      ],
  )
  def hist_kernel(ei_hbm, out_indeg, out_cnt0,
                  src_v, dst_v, ones_v, val_v, zero_v,
                  hist_d, hist_c, sem_a, sem_b):
    c = lax.axis_index("c")
    s = lax.axis_index("s")
    wid = s * 2 + c
    base = wid * _MAIN
    cp_d = pltpu.async_copy(
        ei_hbm.at[1, pl.ds(base, _MAIN)], dst_v.at[pl.ds(0, _MAIN)], sem_b)
    cp_s = pltpu.async_copy(
        ei_hbm.at[0, pl.ds(base, _MAIN)], src_v.at[pl.ds(0, _MAIN)], sem_a)

    @pl.when(wid == _NW - 1)
    def _():
      pltpu.sync_copy(ei_hbm.at[0, pl.ds(_MAIN * _NW, _REM)],
                      src_v.at[pl.ds(_MAIN, _REM)])
      pltpu.sync_copy(ei_hbm.at[1, pl.ds(_MAIN * _NW, _REM)],
                      dst_v.at[pl.ds(_MAIN, _REM)])

    zero16 = jnp.full((16,), 0, jnp.int32)
    one16 = jnp.full((16,), 1, jnp.int32)

    @pl.when(wid != _NW - 1)
    def _():
      dump16 = jnp.full((16,), _DUMP, jnp.int32)

      def pad_body(i, carry):
        sl = pl.ds(_MAIN + i * 16, 16)
        src_v[sl] = zero16
        dst_v[sl] = dump16
        return carry

      lax.fori_loop(0, _REM // 16, pad_body, 0)

    def zero_body(i, carry):
      zero_v[pl.ds(i * 16, 16)] = zero16
      return carry

    lax.fori_loop(0, _ZCH // 16, zero_body, 0)
    pltpu.sync_copy(zero_v, hist_d.at[pl.ds(s * _ZCH, _ZCH)])
    pltpu.sync_copy(zero_v, hist_c.at[pl.ds(s * _ZCH, _ZCH)])

    def ones_body(i, carry):
      ones_v[pl.ds(i * 16, 16)] = one16
      return carry

    lax.fori_loop(0, _FLAT // 16, ones_body, 0)

    cp_d.wait()
    plsc.subcore_barrier()

    sc_d = pltpu.async_copy(ones_v, hist_d.at[dst_v], sem_b, add=True)

    def val_body(i, carry):
      sl = pl.ds(i * 16, 16)
      val_v[sl] = jnp.where(dst_v[sl] == 0, 1, 0).astype(jnp.int32)
      return carry

    lax.fori_loop(0, _FLAT // 16, val_body, 0)

    cp_s.wait()
    sc_c = pltpu.async_copy(val_v, hist_c.at[src_v], sem_a, add=True)
    sc_d.wait()
    sc_c.wait()
    plsc.subcore_barrier()

    @pl.when(s == 0)
    def _():
      pltpu.sync_copy(hist_d, out_indeg.at[c])
      pltpu.sync_copy(hist_c, out_cnt0.at[c])

  return hist_kernel(ei)


def _tc_dense(ind2, cnt2, x, w_emb, b_emb2, w_gcn, b_gcn2, w_cls, b_cls2):
  nb = x.shape[0]
  d = x.shape[1]

  def body(ind_ref, cnt_ref, x_ref, we_ref, be_ref, wg_ref, bg_ref,
           wc_ref, bc_ref, o_ref):
    xx = x_ref[:]
    e0 = jnp.maximum(
        jnp.dot(xx, we_ref[:], preferred_element_type=jnp.float32)
        + be_ref[:], 0.0)
    h1 = jnp.dot(e0, wg_ref[:], preferred_element_type=jnp.float32)
    bg = bg_ref[:]
    h2 = jnp.concatenate([h1[b:b + 1, :] for b in range(nb)], axis=1)
    bg2 = jnp.concatenate([bg] * nb, axis=1)
    h2_hi = h2.astype(jnp.bfloat16).astype(jnp.float32)
    h2_lo = h2 - h2_hi

    ind_full = (ind_ref[0:1, :] + ind_ref[1:2, :]).astype(jnp.float32)
    cnt_full = (cnt_ref[0:1, :] + cnt_ref[1:2, :]).astype(jnp.float32)
    dn = (((0,), (0,)), ((), ()))
    outer = (
        lax.dot_general(ind_full, h2_hi, dn,
                        preferred_element_type=jnp.float32)
        + lax.dot_general(ind_full, h2_lo, dn,
                          preferred_element_type=jnp.float32))
    z = jnp.maximum(outer + bg2, 0.0)
    acc = jnp.dot(cnt_full, z, preferred_element_type=jnp.float32)

    sm = jnp.concatenate(
        [acc[:, b * 128:(b + 1) * 128] for b in range(nb)], axis=0)
    out2 = jnp.maximum(
        jnp.dot(sm, wg_ref[:], preferred_element_type=jnp.float32) + bg, 0.0)
    y = jnp.dot(out2, wc_ref[:], preferred_element_type=jnp.float32) + bc_ref[:]
    o_ref[:] = y

  return pl.pallas_call(
      body,
      out_shape=jax.ShapeDtypeStruct((nb, 1), jnp.float32),
  )(ind2, cnt2, x, w_emb, b_emb2, w_gcn, b_gcn2, w_cls, b_cls2)


def kernel(x, edge_index, W_emb, b_emb, W_gcn, b_gcn, W_cls, b_cls):
  ei = edge_index.astype(jnp.int32)
  ind2, cnt2 = _sc_histograms(ei)
  d = x.shape[1]
  return _tc_dense(
      ind2, cnt2, x, W_emb, b_emb.reshape(1, d), W_gcn,
      b_gcn.reshape(1, d), W_cls, b_cls.reshape(1, 1))


# speedup vs baseline: 1.0510x; 1.0317x over previous
import functools

import jax
import jax.numpy as jnp
from jax import lax
from jax.experimental import pallas as pl
from jax.experimental.pallas import tpu as pltpu
from jax.experimental.pallas import tpu_sc as plsc

_E = 320000
_NW = 32
_MAIN = 9984
_REM = _E - _MAIN * _NW
_FLAT = _MAIN + _REM
_NPAD = 10240
_ZCH = _NPAD // 16
_DUMP = _NPAD - 2


def _sc_histograms(ei):
  mesh = plsc.VectorSubcoreMesh(core_axis_name="c", subcore_axis_name="s")

  @functools.partial(
      pl.kernel,
      out_type=(
          jax.ShapeDtypeStruct((2, _NPAD), jnp.int32),
          jax.ShapeDtypeStruct((2, _NPAD), jnp.int32),
      ),
      mesh=mesh,
      scratch_types=[
          pltpu.VMEM((_FLAT,), jnp.int32),
          pltpu.VMEM((_FLAT,), jnp.int32),
          pltpu.VMEM((_FLAT,), jnp.int32),
          pltpu.VMEM((_FLAT,), jnp.int32),
          pltpu.VMEM((_ZCH,), jnp.int32),
          pltpu.VMEM_SHARED((_NPAD,), jnp.int32),
          pltpu.VMEM_SHARED((_NPAD,), jnp.int32),
          pltpu.SemaphoreType.DMA,
          pltpu.SemaphoreType.DMA,
      ],
  )
  def hist_kernel(ei_hbm, out_indeg, out_cnt0,
                  src_v, dst_v, ones_v, val_v, zero_v,
                  hist_d, hist_c, sem_a, sem_b):
    c = lax.axis_index("c")
    s = lax.axis_index("s")
    wid = s * 2 + c
    base = wid * _MAIN
    cp_d = pltpu.async_copy(
        ei_hbm.at[1, pl.ds(base, _MAIN)], dst_v.at[pl.ds(0, _MAIN)], sem_b)
    cp_s = pltpu.async_copy(
        ei_hbm.at[0, pl.ds(base, _MAIN)], src_v.at[pl.ds(0, _MAIN)], sem_a)

    @pl.when(wid == _NW - 1)
    def _():
      pltpu.sync_copy(ei_hbm.at[0, pl.ds(_MAIN * _NW, _REM)],
                      src_v.at[pl.ds(_MAIN, _REM)])
      pltpu.sync_copy(ei_hbm.at[1, pl.ds(_MAIN * _NW, _REM)],
                      dst_v.at[pl.ds(_MAIN, _REM)])

    zero16 = jnp.full((16,), 0, jnp.int32)
    one16 = jnp.full((16,), 1, jnp.int32)

    @pl.when(wid != _NW - 1)
    def _():
      dump16 = jnp.full((16,), _DUMP, jnp.int32)

      def pad_body(i, carry):
        sl = pl.ds(_MAIN + i * 16, 16)
        src_v[sl] = zero16
        dst_v[sl] = dump16
        return carry

      lax.fori_loop(0, _REM // 16, pad_body, 0)

    def zero_body(i, carry):
      zero_v[pl.ds(i * 16, 16)] = zero16
      return carry

    lax.fori_loop(0, _ZCH // 16, zero_body, 0)
    pltpu.sync_copy(zero_v, hist_d.at[pl.ds(s * _ZCH, _ZCH)])
    pltpu.sync_copy(zero_v, hist_c.at[pl.ds(s * _ZCH, _ZCH)])

    def ones_body(i, carry):
      ones_v[pl.ds(i * 32, 16)] = one16
      ones_v[pl.ds(i * 32 + 16, 16)] = one16
      return carry

    lax.fori_loop(0, _FLAT // 32, ones_body, 0)

    cp_d.wait()
    plsc.subcore_barrier()

    sc_d = pltpu.async_copy(ones_v, hist_d.at[dst_v], sem_b, add=True)

    halfv = _FLAT // 2

    def val_body(i, carry):
      sl = pl.ds(i * 32, 16)
      sl2 = pl.ds(i * 32 + 16, 16)
      val_v[sl] = jnp.where(dst_v[sl] == 0, 1, 0).astype(jnp.int32)
      val_v[sl2] = jnp.where(dst_v[sl2] == 0, 1, 0).astype(jnp.int32)
      return carry

    lax.fori_loop(0, halfv // 32, val_body, 0)
    cp_s.wait()
    sc_c1 = pltpu.async_copy(
        val_v.at[pl.ds(0, halfv)],
        hist_c.at[src_v.at[pl.ds(0, halfv)]], sem_a, add=True)
    lax.fori_loop(halfv // 32, _FLAT // 32, val_body, 0)
    sc_c2 = pltpu.async_copy(
        val_v.at[pl.ds(halfv, _FLAT - halfv)],
        hist_c.at[src_v.at[pl.ds(halfv, _FLAT - halfv)]], sem_a, add=True)
    sc_d.wait()
    sc_c1.wait()
    sc_c2.wait()
    plsc.subcore_barrier()

    @pl.when(s == 0)
    def _():
      pltpu.sync_copy(hist_d, out_indeg.at[c])
      pltpu.sync_copy(hist_c, out_cnt0.at[c])

  return hist_kernel(ei)


def _tc_dense(ind2, cnt2, x, w_emb, b_emb2, w_gcn, b_gcn2, w_cls, b_cls2):
  nb = x.shape[0]
  d = x.shape[1]

  def body(ind_ref, cnt_ref, x_ref, we_ref, be_ref, wg_ref, bg_ref,
           wc_ref, bc_ref, o_ref):
    xx = x_ref[:]
    e0 = jnp.maximum(
        jnp.dot(xx, we_ref[:], preferred_element_type=jnp.float32)
        + be_ref[:], 0.0)
    h1 = jnp.dot(e0, wg_ref[:], preferred_element_type=jnp.float32)
    bg = bg_ref[:]
    h2 = jnp.concatenate([h1[b:b + 1, :] for b in range(nb)], axis=1)
    bg2 = jnp.concatenate([bg] * nb, axis=1)
    h2_hi = h2.astype(jnp.bfloat16).astype(jnp.float32)
    h2_lo = h2 - h2_hi

    ind_full = (ind_ref[0:1, :] + ind_ref[1:2, :]).astype(jnp.float32)
    cnt_full = (cnt_ref[0:1, :] + cnt_ref[1:2, :]).astype(jnp.float32)
    dn = (((0,), (0,)), ((), ()))
    outer = (
        lax.dot_general(ind_full, h2_hi, dn,
                        preferred_element_type=jnp.float32)
        + lax.dot_general(ind_full, h2_lo, dn,
                          preferred_element_type=jnp.float32))
    z = jnp.maximum(outer + bg2, 0.0)
    acc = jnp.dot(cnt_full, z, preferred_element_type=jnp.float32)

    sm = jnp.concatenate(
        [acc[:, b * 128:(b + 1) * 128] for b in range(nb)], axis=0)
    out2 = jnp.maximum(
        jnp.dot(sm, wg_ref[:], preferred_element_type=jnp.float32) + bg, 0.0)
    y = jnp.dot(out2, wc_ref[:], preferred_element_type=jnp.float32) + bc_ref[:]
    o_ref[:] = y

  return pl.pallas_call(
      body,
      out_shape=jax.ShapeDtypeStruct((nb, 1), jnp.float32),
  )(ind2, cnt2, x, w_emb, b_emb2, w_gcn, b_gcn2, w_cls, b_cls2)


def kernel(x, edge_index, W_emb, b_emb, W_gcn, b_gcn, W_cls, b_cls):
  ei = edge_index.astype(jnp.int32)
  ind2, cnt2 = _sc_histograms(ei)
  d = x.shape[1]
  return _tc_dense(
      ind2, cnt2, x, W_emb, b_emb.reshape(1, d), W_gcn,
      b_gcn.reshape(1, d), W_cls, b_cls.reshape(1, 1))


# speedup vs baseline: 1.0673x; 1.0155x over previous
import functools

import jax
import jax.numpy as jnp
from jax import lax
from jax.experimental import pallas as pl
from jax.experimental.pallas import tpu as pltpu
from jax.experimental.pallas import tpu_sc as plsc

_E = 320000
_NW = 32
_MAIN = 9984
_REM = _E - _MAIN * _NW
_FLAT = _MAIN + _REM
_NPAD = 10240
_ZCH = _NPAD // 16
_DUMP = _NPAD - 2


def _sc_histograms(ei):
  mesh = plsc.VectorSubcoreMesh(core_axis_name="c", subcore_axis_name="s")

  @functools.partial(
      pl.kernel,
      out_type=(
          jax.ShapeDtypeStruct((2, _NPAD), jnp.int32),
          jax.ShapeDtypeStruct((2, _NPAD), jnp.int32),
      ),
      mesh=mesh,
      scratch_types=[
          pltpu.VMEM((_FLAT,), jnp.int32),
          pltpu.VMEM((_FLAT,), jnp.int32),
          pltpu.VMEM((_FLAT,), jnp.int32),
          pltpu.VMEM((_FLAT,), jnp.int32),
          pltpu.VMEM((_ZCH,), jnp.int32),
          pltpu.VMEM_SHARED((_NPAD,), jnp.int32),
          pltpu.VMEM_SHARED((_NPAD,), jnp.int32),
          pltpu.SemaphoreType.DMA,
          pltpu.SemaphoreType.DMA,
          pltpu.SemaphoreType.DMA,
      ],
  )
  def hist_kernel(ei_hbm, out_indeg, out_cnt0,
                  src_v, dst_v, ones_v, val_v, zero_v,
                  hist_d, hist_c, sem_a, sem_b, sem_z):
    c = lax.axis_index("c")
    s = lax.axis_index("s")
    wid = s * 2 + c
    base = wid * _MAIN
    cp_d = pltpu.async_copy(
        ei_hbm.at[1, pl.ds(base, _MAIN)], dst_v.at[pl.ds(0, _MAIN)], sem_b)
    cp_s = pltpu.async_copy(
        ei_hbm.at[0, pl.ds(base, _MAIN)], src_v.at[pl.ds(0, _MAIN)], sem_a)

    @pl.when(wid == _NW - 1)
    def _():
      pltpu.sync_copy(ei_hbm.at[0, pl.ds(_MAIN * _NW, _REM)],
                      src_v.at[pl.ds(_MAIN, _REM)])
      pltpu.sync_copy(ei_hbm.at[1, pl.ds(_MAIN * _NW, _REM)],
                      dst_v.at[pl.ds(_MAIN, _REM)])

    zero16 = jnp.full((16,), 0, jnp.int32)
    one16 = jnp.full((16,), 1, jnp.int32)

    @pl.when(wid != _NW - 1)
    def _():
      dump16 = jnp.full((16,), _DUMP, jnp.int32)

      def pad_body(i, carry):
        sl = pl.ds(_MAIN + i * 16, 16)
        src_v[sl] = zero16
        dst_v[sl] = dump16
        return carry

      lax.fori_loop(0, _REM // 16, pad_body, 0)

    def zero_body(i, carry):
      zero_v[pl.ds(i * 32, 16)] = zero16
      zero_v[pl.ds(i * 32 + 16, 16)] = zero16
      return carry

    lax.fori_loop(0, _ZCH // 32, zero_body, 0)
    zc_d = pltpu.async_copy(zero_v, hist_d.at[pl.ds(s * _ZCH, _ZCH)], sem_z)
    zc_c = pltpu.async_copy(zero_v, hist_c.at[pl.ds(s * _ZCH, _ZCH)], sem_z)

    def ones_body(i, carry):
      for j in range(4):
        ones_v[pl.ds(i * 64 + j * 16, 16)] = one16
      return carry

    lax.fori_loop(0, _FLAT // 64, ones_body, 0)
    zc_d.wait()
    zc_c.wait()

    cp_d.wait()
    plsc.subcore_barrier()

    sc_d = pltpu.async_copy(ones_v, hist_d.at[dst_v], sem_b, add=True)

    halfv = _FLAT // 2

    def val_body(i, carry):
      for j in range(4):
        sl = pl.ds(i * 64 + j * 16, 16)
        val_v[sl] = jnp.where(dst_v[sl] == 0, 1, 0).astype(jnp.int32)
      return carry

    lax.fori_loop(0, halfv // 64, val_body, 0)
    cp_s.wait()
    sc_c1 = pltpu.async_copy(
        val_v.at[pl.ds(0, halfv)],
        hist_c.at[src_v.at[pl.ds(0, halfv)]], sem_a, add=True)
    lax.fori_loop(halfv // 64, _FLAT // 64, val_body, 0)
    sc_c2 = pltpu.async_copy(
        val_v.at[pl.ds(halfv, _FLAT - halfv)],
        hist_c.at[src_v.at[pl.ds(halfv, _FLAT - halfv)]], sem_a, add=True)
    sc_d.wait()
    sc_c1.wait()
    sc_c2.wait()
    plsc.subcore_barrier()

    @pl.when(s == 0)
    def _():
      pltpu.sync_copy(hist_d, out_indeg.at[c])
      pltpu.sync_copy(hist_c, out_cnt0.at[c])

  return hist_kernel(ei)


def _tc_dense(ind2, cnt2, x, w_emb, b_emb2, w_gcn, b_gcn2, w_cls, b_cls2):
  nb = x.shape[0]
  d = x.shape[1]

  def body(ind_ref, cnt_ref, x_ref, we_ref, be_ref, wg_ref, bg_ref,
           wc_ref, bc_ref, o_ref):
    xx = x_ref[:]
    e0 = jnp.maximum(
        jnp.dot(xx, we_ref[:], preferred_element_type=jnp.float32)
        + be_ref[:], 0.0)
    h1 = jnp.dot(e0, wg_ref[:], preferred_element_type=jnp.float32)
    bg = bg_ref[:]
    h2 = jnp.concatenate([h1[b:b + 1, :] for b in range(nb)], axis=1)
    bg2 = jnp.concatenate([bg] * nb, axis=1)
    h2_hi = h2.astype(jnp.bfloat16).astype(jnp.float32)
    h2_lo = h2 - h2_hi

    ind_full = (ind_ref[0:1, :] + ind_ref[1:2, :]).astype(jnp.float32)
    cnt_full = (cnt_ref[0:1, :] + cnt_ref[1:2, :]).astype(jnp.float32)
    dn = (((0,), (0,)), ((), ()))
    outer = (
        lax.dot_general(ind_full, h2_hi, dn,
                        preferred_element_type=jnp.float32)
        + lax.dot_general(ind_full, h2_lo, dn,
                          preferred_element_type=jnp.float32))
    z = jnp.maximum(outer + bg2, 0.0)
    acc = jnp.dot(cnt_full, z, preferred_element_type=jnp.float32)

    sm = jnp.concatenate(
        [acc[:, b * 128:(b + 1) * 128] for b in range(nb)], axis=0)
    out2 = jnp.maximum(
        jnp.dot(sm, wg_ref[:], preferred_element_type=jnp.float32) + bg, 0.0)
    y = jnp.dot(out2, wc_ref[:], preferred_element_type=jnp.float32) + bc_ref[:]
    o_ref[:] = y

  return pl.pallas_call(
      body,
      out_shape=jax.ShapeDtypeStruct((nb, 1), jnp.float32),
  )(ind2, cnt2, x, w_emb, b_emb2, w_gcn, b_gcn2, w_cls, b_cls2)


def kernel(x, edge_index, W_emb, b_emb, W_gcn, b_gcn, W_cls, b_cls):
  ei = edge_index.astype(jnp.int32)
  ind2, cnt2 = _sc_histograms(ei)
  d = x.shape[1]
  return _tc_dense(
      ind2, cnt2, x, W_emb, b_emb.reshape(1, d), W_gcn,
      b_gcn.reshape(1, d), W_cls, b_cls.reshape(1, 1))
